# Initial kernel scaffold; baseline (speedup 1.0000x reference)
import functools

import jax
import jax.numpy as jnp
from jax.experimental import pallas as pl
from jax.experimental.pallas import tpu as pltpu
from jax.experimental.pallas import tpu_sc as plsc

IN_F = 4096
OUT_F = 1024
BATCH = 16384
ROWS_PER_STEP = 8
LANES = 16


def _argmax_body(w_ref, o_ref):
    w = w_ref[...]
    m = jnp.max(w, axis=1, keepdims=True)
    ii = jax.lax.broadcasted_iota(jnp.int32, w.shape, 1)
    cand = jnp.where(w == m, ii, IN_F)
    o_ref[0, 0, :] = jnp.min(cand, axis=1).astype(jnp.int32)


def _argmax(raw_weight):
    out = pl.pallas_call(
        _argmax_body,
        grid=(OUT_F // 128,),
        in_specs=[pl.BlockSpec((128, IN_F), lambda i: (i, 0))],
        out_specs=pl.BlockSpec((1, 1, 128), lambda i: (i, 0, 0)),
        out_shape=jax.ShapeDtypeStruct((OUT_F // 128, 1, 128), jnp.int32),
    )(raw_weight)
    return out.reshape(1, OUT_F)


def _gather_sc(x, idx):
    mesh = plsc.VectorSubcoreMesh(core_axis_name="c", subcore_axis_name="s")

    @functools.partial(
        pl.kernel,
        mesh=mesh,
        out_type=jax.ShapeDtypeStruct((BATCH, OUT_F), jnp.float32),
    )
    def k(i_hbm, x_hbm, o_hbm):
        def body(i_vmem, x_vmem, o_vmem):
            idx_ref = i_vmem.at[0]

            @pl.loop(0, OUT_F // LANES)
            def _(j):
                cols = idx_ref[pl.ds(j * LANES, LANES)]
                for r in range(ROWS_PER_STEP):
                    rows = jnp.full((LANES,), r, jnp.int32)
                    vals = plsc.load_gather(x_vmem, [rows, cols])
                    o_vmem[r, pl.ds(j * LANES, LANES)] = vals

        pltpu.emit_pipeline(
            body,
            grid=(BATCH // ROWS_PER_STEP,),
            in_specs=[
                pl.BlockSpec((1, OUT_F), lambda i: (0, 0)),
                pl.BlockSpec((ROWS_PER_STEP, IN_F), lambda i: (i, 0)),
            ],
            out_specs=[pl.BlockSpec((ROWS_PER_STEP, OUT_F), lambda i: (i, 0))],
            core_axis_name=("c", "s"),
            dimension_semantics=(pltpu.PARALLEL,),
        )(i_hbm, x_hbm, o_hbm)

    return k(idx, x)


def kernel(x, raw_weight):
    idx = _argmax(raw_weight)
    return _gather_sc(x, idx)


# speedup vs baseline: 1.9603x; 1.9603x over previous
import dataclasses
import functools

import jax
import jax.numpy as jnp
from jax.experimental import pallas as pl
from jax.experimental.pallas import tpu as pltpu
from jax.experimental.pallas import tpu_sc as plsc

IN_F = 4096
OUT_F = 1024
BATCH = 16384
ROWS_PER_STEP = 8
LANES = 16


def _argmax_body(w_ref, o_ref):
    w = w_ref[...]
    m = jnp.max(w, axis=1, keepdims=True)
    ii = jax.lax.broadcasted_iota(jnp.int32, w.shape, 1)
    cand = jnp.where(w == m, ii, IN_F)
    o_ref[0, 0, :] = jnp.min(cand, axis=1).astype(jnp.int32)


def _argmax(raw_weight):
    out = pl.pallas_call(
        _argmax_body,
        grid=(OUT_F // 128,),
        in_specs=[pl.BlockSpec((128, IN_F), lambda i: (i, 0))],
        out_specs=pl.BlockSpec((1, 1, 128), lambda i: (i, 0, 0)),
        out_shape=jax.ShapeDtypeStruct((OUT_F // 128, 1, 128), jnp.int32),
    )(raw_weight)
    return out.reshape(1, OUT_F)


def _gather_sc(x, idx):
    mesh = plsc.VectorSubcoreMesh(core_axis_name="c", subcore_axis_name="s")
    cp = pltpu.CompilerParams()
    if "needs_layout_passes" in pltpu.CompilerParams.__dataclass_fields__:
        cp = dataclasses.replace(cp, needs_layout_passes=False)

    @functools.partial(
        pl.kernel,
        mesh=mesh,
        out_type=jax.ShapeDtypeStruct((BATCH, OUT_F), jnp.float32),
        compiler_params=cp,
    )
    def k(i_hbm, x_hbm, o_hbm):
        def body(i_vmem, x_vmem, o_vmem):
            idx_ref = i_vmem.at[0]

            @pl.loop(0, OUT_F // LANES)
            def _(j):
                cols = idx_ref[pl.ds(j * LANES, LANES)]
                for r in range(ROWS_PER_STEP):
                    rows = jnp.full((LANES,), r, jnp.int32)
                    vals = plsc.load_gather(x_vmem, [rows, cols])
                    o_vmem[r, pl.ds(j * LANES, LANES)] = vals

        pltpu.emit_pipeline(
            body,
            grid=(BATCH // ROWS_PER_STEP,),
            in_specs=[
                pl.BlockSpec((1, OUT_F), lambda i: (0, 0)),
                pl.BlockSpec((ROWS_PER_STEP, IN_F), lambda i: (i, 0)),
            ],
            out_specs=[pl.BlockSpec((ROWS_PER_STEP, OUT_F), lambda i: (i, 0))],
            core_axis_name=("c", "s"),
            dimension_semantics=(pltpu.PARALLEL,),
        )(i_hbm, x_hbm, o_hbm)

    return k(idx, x)


def kernel(x, raw_weight):
    idx = _argmax(raw_weight)
    return _gather_sc(x, idx)
